# baseline (speedup 1.0000x reference)
import functools

import jax
import jax.numpy as jnp
from jax import lax
from jax.experimental import pallas as pl
from jax.experimental.pallas import tpu as pltpu
from jax.experimental.pallas import tpu_sc as plsc

N = 10000
E = 320000
D = 128

NC = 2
NS = 16
CH = 128
KC = 160
Q = 16
EPS = KC * CH
E_PAD = NS * EPS
NPH = 5120
NPC = NPH + 128
RPT = NPC // NS
DUMMY = NPH
NPT = NC * NPH
BAND = 48
NB = NS * BAND


def _sc_aggregate():
    mesh = plsc.VectorSubcoreMesh(
        core_axis_name="c", subcore_axis_name="s", num_cores=NC, num_subcores=NS
    )

    @functools.partial(
        pl.kernel,
        out_type=[
            jax.ShapeDtypeStruct((NC, NPC, D), jnp.float32),
            jax.ShapeDtypeStruct((NC, NB, D), jnp.float32),
        ],
        mesh=mesh,
        scratch_types=[
            pltpu.VMEM((Q, CH), jnp.int32),
            pltpu.VMEM((Q, CH), jnp.int32),
            pltpu.VMEM((2, CH), jnp.int32),
            pltpu.VMEM((2, CH), jnp.int32),
            pltpu.VMEM((2, CH), jnp.int32),
            pltpu.VMEM((2, CH), jnp.int32),
            pltpu.VMEM((CH, D), jnp.float32),
            pltpu.VMEM((CH, D), jnp.float32),
            pltpu.VMEM((CH, D), jnp.float32),
            pltpu.VMEM((CH, D), jnp.float32),
            pltpu.VMEM_SHARED((NPC, D), jnp.float32),
            pltpu.VMEM_SHARED((NB, D), jnp.float32),
            pltpu.SemaphoreType.DMA,
            pltpu.SemaphoreType.DMA,
            pltpu.SemaphoreType.DMA,
            pltpu.SemaphoreType.DMA,
        ],
    )
    def body(feat_ext, srcp, dstp, zf, zb, acc_out, deg_out,
             src_blk, dst_blk, ssrc, sdst, ohi, dhi,
             rows0, rows1, oh0, oh1, acc, dacc, gs0, gs1, ss0, ss1):
        c = lax.axis_index("c")
        s = lax.axis_index("s")
        base = s * RPT
        rowsb = (rows0, rows1)
        ohb = (oh0, oh1)
        gsem = (gs0, gs1)
        ssem = (ss0, ss1)

        pltpu.sync_copy(zf, acc.at[pl.ds(base, RPT)])
        pltpu.sync_copy(zb, dacc.at[pl.ds(s * BAND, BAND)])
        plsc.subcore_barrier()

        def load_blk(b):
            pltpu.sync_copy(srcp.at[c, s, pl.ds(b * Q, Q)], src_blk)
            pltpu.sync_copy(dstp.at[c, s, pl.ds(b * Q, Q)], dst_blk)

        def calc(q, p):
            for g in range(CH // 16):
                s16 = src_blk[q, pl.ds(g * 16, 16)]
                d16 = dst_blk[q, pl.ds(g * 16, 16)]
                ssrc[p, pl.ds(g * 16, 16)] = s16
                sdst[p, pl.ds(g * 16, 16)] = d16
                ohi[p, pl.ds(g * 16, 16)] = N + (d16 & 127)
                dhi[p, pl.ds(g * 16, 16)] = (
                    lax.shift_right_logical(d16, 7) + s * BAND
                )

        def fire_gath(p):
            pltpu.async_copy(feat_ext.at[ssrc.at[p]], rowsb[p], gsem[p])
            pltpu.async_copy(feat_ext.at[ohi.at[p]], ohb[p], gsem[p])

        def drain_gath(p):
            pltpu.make_async_copy(
                feat_ext.at[pl.ds(0, CH)], rowsb[p], gsem[p]
            ).wait()
            pltpu.make_async_copy(
                feat_ext.at[pl.ds(0, CH)], ohb[p], gsem[p]
            ).wait()

        def fire_scat(p):
            pltpu.async_copy(rowsb[p], acc.at[sdst.at[p]], ssem[p], add=True)
            pltpu.async_copy(ohb[p], dacc.at[dhi.at[p]], ssem[p], add=True)

        def drain_scat(p):
            pltpu.make_async_copy(
                rowsb[p], acc.at[pl.ds(0, CH)], ssem[p]
            ).wait()
            pltpu.make_async_copy(
                ohb[p], dacc.at[pl.ds(0, CH)], ssem[p]
            ).wait()

        load_blk(0)
        calc(0, 0)
        fire_gath(0)
        for q in range(1, Q):
            p = q & 1
            if q >= 2:
                drain_scat(p)
            calc(q, p)
            fire_gath(p)
            drain_gath(1 - p)
            fire_scat(1 - p)

        def blk(b, carry):
            load_blk(b)
            for q in range(Q):
                p = q & 1
                drain_scat(p)
                calc(q, p)
                fire_gath(p)
                drain_gath(1 - p)
                fire_scat(1 - p)
            return carry

        lax.fori_loop(1, KC // Q, blk, 0)

        drain_gath(1)
        fire_scat(1)
        drain_scat(0)
        drain_scat(1)
        plsc.subcore_barrier()

        pltpu.sync_copy(acc.at[pl.ds(base, RPT)], acc_out.at[c, pl.ds(base, RPT)])
        pltpu.sync_copy(dacc.at[pl.ds(s * BAND, BAND)],
                        deg_out.at[c, pl.ds(s * BAND, BAND)])

    return body


def _tc_body(a_ref, d_ref, x_ref, wt_ref, b_ref, o_ref):
    deg = jnp.sum(d_ref[...], axis=0)[:, None] + 1.0
    h = (a_ref[...] + x_ref[...]) / deg
    o_ref[...] = (
        jnp.dot(h, wt_ref[...], preferred_element_type=jnp.float32) + b_ref[...]
    )


def kernel(node_feat, edge_index, W_neigh, b):
    src = edge_index[0]
    dst = edge_index[1]
    pad = E_PAD - E
    spread = (jnp.arange(pad, dtype=jnp.int32) % 128)
    srcf = jnp.concatenate([src, spread])
    dstf = jnp.concatenate([dst, jnp.asarray(N, jnp.int32) + spread])
    dsth, ssth = [], []
    for c in range(NC):
        local = dstf - c * NPH
        inr = (local >= 0) & (local < NPH)
        dloc = jnp.where(inr, local, DUMMY + (srcf % 128))
        sloc = jnp.where(inr, srcf, srcf & 127)
        dsth.append(dloc.reshape(NS, KC, CH))
        ssth.append(sloc.reshape(NS, KC, CH))
    dstp = jnp.stack(dsth)
    srcp = jnp.stack(ssth)
    feat_ext = jnp.concatenate([node_feat, jnp.eye(D, dtype=jnp.float32)], axis=0)
    zf = jnp.zeros((RPT, D), jnp.float32)
    zb = jnp.zeros((BAND, D), jnp.float32)

    acc_part, deg_part = _sc_aggregate()(feat_ext, srcp, dstp, zf, zb)

    agg = acc_part[:, :NPH, :].reshape(NPT, D)
    degs = (
        deg_part.reshape(NC, NS, BAND * D)[:, :, : NPC]
        [:, :, :NPH]
        .transpose(1, 0, 2)
        .reshape(NS, NPT)
    )
    xp = jnp.concatenate([node_feat, jnp.zeros((NPT - N, D), jnp.float32)], axis=0)
    wt = W_neigh.T
    b2 = b.reshape(1, D)

    BLK = 256
    out = pl.pallas_call(
        _tc_body,
        grid=(NPT // BLK,),
        in_specs=[
            pl.BlockSpec((BLK, D), lambda i: (i, 0)),
            pl.BlockSpec((NS, BLK), lambda i: (0, i)),
            pl.BlockSpec((BLK, D), lambda i: (i, 0)),
            pl.BlockSpec((D, D), lambda i: (0, 0)),
            pl.BlockSpec((1, D), lambda i: (0, 0)),
        ],
        out_specs=pl.BlockSpec((BLK, D), lambda i: (i, 0)),
        out_shape=jax.ShapeDtypeStruct((NPT, D), jnp.float32),
    )(agg, degs, xp, wt, b2)
    return out[:N]


# speedup vs baseline: 1.5349x; 1.5349x over previous
import functools

import jax
import jax.numpy as jnp
from jax import lax
from jax.experimental import pallas as pl
from jax.experimental.pallas import tpu as pltpu
from jax.experimental.pallas import tpu_sc as plsc

N = 10000
E = 320000
D = 128

NC = 2
NS = 16
CH = 128
KC = 160
Q = 16
EPS = KC * CH
E_PAD = NS * EPS
NPH = 5120
NPC = NPH + 128
RPT = NPC // NS
DUMMY = NPH
NPT = NC * NPH
BAND = 48
NB = NS * BAND


def _sc_aggregate():
    mesh = plsc.VectorSubcoreMesh(
        core_axis_name="c", subcore_axis_name="s", num_cores=NC, num_subcores=NS
    )

    @functools.partial(
        pl.kernel,
        out_type=[
            jax.ShapeDtypeStruct((NC, NPC, D), jnp.float32),
            jax.ShapeDtypeStruct((NC, NB, D), jnp.float32),
        ],
        mesh=mesh,
        scratch_types=[
            pltpu.VMEM((Q, CH), jnp.int32),
            pltpu.VMEM((Q, CH), jnp.int32),
            pltpu.VMEM((2, CH), jnp.int32),
            pltpu.VMEM((2, CH), jnp.int32),
            pltpu.VMEM((2, CH), jnp.int32),
            pltpu.VMEM((2, CH), jnp.int32),
            pltpu.VMEM((CH, D), jnp.float32),
            pltpu.VMEM((CH, D), jnp.float32),
            pltpu.VMEM((CH, D), jnp.float32),
            pltpu.VMEM((CH, D), jnp.float32),
            pltpu.VMEM_SHARED((NPC, D), jnp.float32),
            pltpu.VMEM_SHARED((NB, D), jnp.float32),
            pltpu.SemaphoreType.DMA,
            pltpu.SemaphoreType.DMA,
            pltpu.SemaphoreType.DMA,
            pltpu.SemaphoreType.DMA,
        ],
    )
    def body(feat_ext, srcp, dstp, zf, zb, acc_out, deg_out,
             src_blk, dst_blk, ssrc, sdst, ohi, dhi,
             rows0, rows1, oh0, oh1, acc, dacc, gs0, gs1, ss0, ss1):
        c = lax.axis_index("c")
        s = lax.axis_index("s")
        base = s * RPT
        rowsb = (rows0, rows1)
        ohb = (oh0, oh1)
        gsem = (gs0, gs1)
        ssem = (ss0, ss1)

        pltpu.sync_copy(zf, acc.at[pl.ds(base, RPT)])
        pltpu.sync_copy(zb, dacc.at[pl.ds(s * BAND, BAND)])
        plsc.subcore_barrier()

        def load_blk(b):
            pltpu.sync_copy(srcp.at[s, pl.ds(b * Q, Q)], src_blk)
            pltpu.sync_copy(dstp.at[c, s, pl.ds(b * Q, Q)], dst_blk)

        def calc(q, p):
            for g in range(CH // 16):
                s16 = src_blk[q, pl.ds(g * 16, 16)]
                d16 = dst_blk[q, pl.ds(g * 16, 16)]
                ssrc[p, pl.ds(g * 16, 16)] = s16
                sdst[p, pl.ds(g * 16, 16)] = d16
                ohi[p, pl.ds(g * 16, 16)] = N + s * 128 + (d16 & 127)
                dhi[p, pl.ds(g * 16, 16)] = (
                    lax.shift_right_logical(d16, 7) + s * BAND
                )

        def fire_gath(p):
            pltpu.async_copy(feat_ext.at[ssrc.at[p]], rowsb[p], gsem[p])
            pltpu.async_copy(feat_ext.at[ohi.at[p]], ohb[p], gsem[p])

        def drain_gath(p):
            pltpu.make_async_copy(
                feat_ext.at[pl.ds(0, CH)], rowsb[p], gsem[p]
            ).wait()
            pltpu.make_async_copy(
                feat_ext.at[pl.ds(0, CH)], ohb[p], gsem[p]
            ).wait()

        def fire_scat(p):
            pltpu.async_copy(rowsb[p], acc.at[sdst.at[p]], ssem[p], add=True)
            pltpu.async_copy(ohb[p], dacc.at[dhi.at[p]], ssem[p], add=True)

        def drain_scat(p):
            pltpu.make_async_copy(
                rowsb[p], acc.at[pl.ds(0, CH)], ssem[p]
            ).wait()
            pltpu.make_async_copy(
                ohb[p], dacc.at[pl.ds(0, CH)], ssem[p]
            ).wait()

        load_blk(0)
        calc(0, 0)
        fire_gath(0)
        for q in range(1, Q):
            p = q & 1
            if q >= 2:
                drain_scat(p)
            calc(q, p)
            fire_gath(p)
            drain_gath(1 - p)
            fire_scat(1 - p)

        def blk(b, carry):
            load_blk(b)
            for q in range(Q):
                p = q & 1
                drain_scat(p)
                calc(q, p)
                fire_gath(p)
                drain_gath(1 - p)
                fire_scat(1 - p)
            return carry

        lax.fori_loop(1, KC // Q, blk, 0)

        drain_gath(1)
        fire_scat(1)
        drain_scat(0)
        drain_scat(1)
        plsc.subcore_barrier()

        pltpu.sync_copy(acc.at[pl.ds(base, RPT)], acc_out.at[c, pl.ds(base, RPT)])
        pltpu.sync_copy(dacc.at[pl.ds(s * BAND, BAND)],
                        deg_out.at[c, pl.ds(s * BAND, BAND)])

    return body


def _tc_body(a_ref, d_ref, x_ref, wt_ref, b_ref, o_ref):
    deg = jnp.sum(d_ref[...], axis=0)[:, None] + 1.0
    h = (a_ref[...] + x_ref[...]) / deg
    o_ref[...] = (
        jnp.dot(h, wt_ref[...], preferred_element_type=jnp.float32) + b_ref[...]
    )


def kernel(node_feat, edge_index, W_neigh, b):
    src = edge_index[0]
    dst = edge_index[1]
    pad = E_PAD - E
    spread = (jnp.arange(pad, dtype=jnp.int32) % 128)
    srcf = jnp.concatenate([src, spread])
    dstf = jnp.concatenate([dst, jnp.asarray(N, jnp.int32) + spread])
    srcp = srcf.reshape(NS, KC, CH)
    dsth = []
    for c in range(NC):
        local = dstf - c * NPH
        inr = (local >= 0) & (local < NPH)
        dloc = jnp.where(inr, local, DUMMY + (srcf % 128))
        dsth.append(dloc.reshape(NS, KC, CH))
    dstp = jnp.stack(dsth)
    eye_rep = jnp.tile(jnp.eye(D, dtype=jnp.float32), (NS, 1))
    feat_ext = jnp.concatenate([node_feat, eye_rep], axis=0)
    zf = jnp.zeros((RPT, D), jnp.float32)
    zb = jnp.zeros((BAND, D), jnp.float32)

    acc_part, deg_part = _sc_aggregate()(feat_ext, srcp, dstp, zf, zb)

    agg = acc_part[:, :NPH, :].reshape(NPT, D)
    degs = (
        deg_part.reshape(NC, NS, BAND * D)[:, :, : NPC]
        [:, :, :NPH]
        .transpose(1, 0, 2)
        .reshape(NS, NPT)
    )
    xp = jnp.concatenate([node_feat, jnp.zeros((NPT - N, D), jnp.float32)], axis=0)
    wt = W_neigh.T
    b2 = b.reshape(1, D)

    BLK = 256
    out = pl.pallas_call(
        _tc_body,
        grid=(NPT // BLK,),
        in_specs=[
            pl.BlockSpec((BLK, D), lambda i: (i, 0)),
            pl.BlockSpec((NS, BLK), lambda i: (0, i)),
            pl.BlockSpec((BLK, D), lambda i: (i, 0)),
            pl.BlockSpec((D, D), lambda i: (0, 0)),
            pl.BlockSpec((1, D), lambda i: (0, 0)),
        ],
        out_specs=pl.BlockSpec((BLK, D), lambda i: (i, 0)),
        out_shape=jax.ShapeDtypeStruct((NPT, D), jnp.float32),
    )(agg, degs, xp, wt, b2)
    return out[:N]
